# baseline (speedup 1.0000x reference)
import functools

import jax
import jax.numpy as jnp
from jax import lax
from jax.experimental import pallas as pl
from jax.experimental.pallas import tpu as pltpu
from jax.experimental.pallas import tpu_sc as plsc

_DIM = 64
_SCALE = 10.0

_info = plsc.get_sparse_core_info()
_NC, _NS = _info.num_cores, _info.num_subcores
_NW = _NC * _NS

_CH = 200
_G1 = 104


@functools.lru_cache(maxsize=None)
def _make_gather(b_total):
    assert b_total % (_NW * _CH) == 0
    n_it = b_total // (_NW * _CH)
    assert n_it % 2 == 0
    b_per_w = n_it * _CH
    g2 = _CH - _G1
    mesh = plsc.VectorSubcoreMesh(core_axis_name="c", subcore_axis_name="s")

    @functools.partial(
        pl.kernel,
        mesh=mesh,
        out_type=jax.ShapeDtypeStruct((b_total, 2 * _DIM), jnp.float32),
        compiler_params=pltpu.CompilerParams(use_tc_tiling_on_sc=False),
        scratch_types=[
            pltpu.VMEM((b_per_w,), jnp.int32),
            pltpu.VMEM((2, _CH, _DIM), jnp.float32),
            pltpu.VMEM((2, _CH, 2 * _DIM), jnp.float32),
            pltpu.SemaphoreType.DMA,
            pltpu.SemaphoreType.DMA,
            pltpu.SemaphoreType.DMA,
            pltpu.SemaphoreType.DMA,
        ],
    )
    def k(table_hbm, idx_hbm, out_hbm, idx_v, rows, wide, gsem0, gsem1,
          osem0, osem1):
        gsems = (gsem0, gsem1)
        osems = (osem0, osem1)
        wid = lax.axis_index("s") * _NC + lax.axis_index("c")
        base = wid * b_per_w
        pltpu.sync_copy(idx_hbm.at[pl.ds(base, b_per_w)], idx_v)

        def fire_gathers(j, b):
            off = j * _CH
            pltpu.async_copy(
                table_hbm.at[idx_v.at[pl.ds(off, _G1)]],
                rows.at[b, pl.ds(0, _G1)],
                gsems[b],
            )
            pltpu.async_copy(
                table_hbm.at[idx_v.at[pl.ds(off + _G1, g2)]],
                rows.at[b, pl.ds(_G1, g2)],
                gsems[b],
            )

        def wait_gathers(b):
            pltpu.make_async_copy(
                table_hbm.at[idx_v.at[pl.ds(0, _G1)]],
                rows.at[b, pl.ds(0, _G1)],
                gsems[b],
            ).wait()
            pltpu.make_async_copy(
                table_hbm.at[idx_v.at[pl.ds(0, g2)]],
                rows.at[b, pl.ds(_G1, g2)],
                gsems[b],
            ).wait()

        fire_gathers(0, 0)

        def grp(g, carry):
            for b in range(2):
                j = g * 2 + b
                nb = 1 - b
                wait_gathers(b)

                @pl.when(j + 1 < n_it)
                def _():
                    fire_gathers(j + 1, nb)

                @pl.when(j >= 2)
                def _():
                    pltpu.make_async_copy(
                        wide.at[b], out_hbm.at[pl.ds(0, _CH)], osems[b]
                    ).wait()

                @plsc.parallel_loop(0, _CH * _DIM // 16, step=1, unroll=8)
                def _(p):
                    r = p // (_DIM // 16)
                    c = (p % (_DIM // 16)) * 16
                    wide[b, r, pl.ds(c, 16)] = (
                        rows[b, r, pl.ds(c, 16)] * _SCALE
                    )

                pltpu.async_copy(
                    wide.at[b],
                    out_hbm.at[pl.ds(base + j * _CH, _CH)],
                    osems[b],
                )

            return carry

        lax.fori_loop(0, n_it // 2, grp, 0)
        for b in range(2):
            pltpu.make_async_copy(
                wide.at[b], out_hbm.at[pl.ds(0, _CH)], osems[b]
            ).wait()

    return k


def _fmt_body(x_ref, o_ref):
    o_ref[...] = x_ref[:, :_DIM].reshape(o_ref.shape)


@functools.lru_cache(maxsize=None)
def _make_format(n_rows, seq):
    bo = 128
    assert n_rows % bo == 0
    return pl.pallas_call(
        _fmt_body,
        grid=(n_rows // bo,),
        in_specs=[pl.BlockSpec((bo * seq, 2 * _DIM), lambda i: (i, 0))],
        out_specs=pl.BlockSpec((bo, seq, _DIM), lambda i: (i, 0, 0)),
        out_shape=jax.ShapeDtypeStruct((n_rows, seq, _DIM), jnp.float32),
    )


def kernel(inputs, table):
    n_rows, seq = inputs.shape
    idx = inputs.reshape(inputs.size)
    wide = _make_gather(inputs.size)(table, idx)
    return _make_format(n_rows, seq)(wide)


# speedup vs baseline: 1.2776x; 1.2776x over previous
import functools

import jax
import jax.numpy as jnp
from jax import lax
from jax.experimental import pallas as pl
from jax.experimental.pallas import tpu as pltpu
from jax.experimental.pallas import tpu_sc as plsc

_DIM = 64
_SCALE = 10.0

_info = plsc.get_sparse_core_info()
_NC, _NS = _info.num_cores, _info.num_subcores
_NW = _NC * _NS

_OC = 4
_G1 = 104


@functools.lru_cache(maxsize=None)
def _make_gather(n_rows, seq):
    flat_per_chunk = _OC * seq
    assert n_rows % _NW == 0
    rows_per_w = n_rows // _NW
    assert rows_per_w % _OC == 0
    n_it = rows_per_w // _OC
    assert n_it % 2 == 0
    b_per_w = rows_per_w * seq
    g2 = flat_per_chunk - _G1
    mesh = plsc.VectorSubcoreMesh(core_axis_name="c", subcore_axis_name="s")

    @functools.partial(
        pl.kernel,
        mesh=mesh,
        out_type=jax.ShapeDtypeStruct((n_rows, seq, _DIM), jnp.float32),
        compiler_params=pltpu.CompilerParams(use_tc_tiling_on_sc=True),
        scratch_types=[
            pltpu.VMEM((b_per_w,), jnp.int32),
            pltpu.VMEM((2, flat_per_chunk, 2 * _DIM), jnp.float32),
            pltpu.VMEM((2, _OC, seq, _DIM), jnp.float32),
            pltpu.SemaphoreType.DMA,
            pltpu.SemaphoreType.DMA,
            pltpu.SemaphoreType.DMA,
            pltpu.SemaphoreType.DMA,
        ],
    )
    def k(table_hbm, idx_hbm, out_hbm, idx_v, rows, boxes, gsem0, gsem1,
          osem0, osem1):
        gsems = (gsem0, gsem1)
        osems = (osem0, osem1)
        wid = lax.axis_index("s") * _NC + lax.axis_index("c")
        base = wid * b_per_w
        row0 = wid * rows_per_w
        pltpu.sync_copy(idx_hbm.at[pl.ds(base, b_per_w)], idx_v)

        def fire_gathers(j, b):
            off = j * flat_per_chunk
            pltpu.async_copy(
                table_hbm.at[idx_v.at[pl.ds(off, _G1)]],
                rows.at[b, pl.ds(0, _G1)],
                gsems[b],
            )
            pltpu.async_copy(
                table_hbm.at[idx_v.at[pl.ds(off + _G1, g2)]],
                rows.at[b, pl.ds(_G1, g2)],
                gsems[b],
            )

        def wait_gathers(b):
            pltpu.make_async_copy(
                table_hbm.at[idx_v.at[pl.ds(0, _G1)]],
                rows.at[b, pl.ds(0, _G1)],
                gsems[b],
            ).wait()
            pltpu.make_async_copy(
                table_hbm.at[idx_v.at[pl.ds(0, g2)]],
                rows.at[b, pl.ds(_G1, g2)],
                gsems[b],
            ).wait()

        fire_gathers(0, 0)

        def grp(g, carry):
            for b in range(2):
                j = g * 2 + b
                nb = 1 - b
                wait_gathers(b)

                @pl.when(j + 1 < n_it)
                def _():
                    fire_gathers(j + 1, nb)

                @pl.when(j >= 2)
                def _():
                    pltpu.make_async_copy(
                        boxes.at[b], out_hbm.at[pl.ds(0, _OC)], osems[b]
                    ).wait()

                for o in range(_OC):
                    @plsc.parallel_loop(0, seq * _DIM // 16, step=1, unroll=8)
                    def _(p):
                        r = p // (_DIM // 16)
                        c = (p % (_DIM // 16)) * 16
                        boxes[b, o, r, pl.ds(c, 16)] = (
                            rows[b, o * seq + r, pl.ds(c, 16)] * _SCALE
                        )

                pltpu.async_copy(
                    boxes.at[b],
                    out_hbm.at[pl.ds(row0 + j * _OC, _OC)],
                    osems[b],
                )

            return carry

        lax.fori_loop(0, n_it // 2, grp, 0)
        for b in range(2):
            pltpu.make_async_copy(
                boxes.at[b], out_hbm.at[pl.ds(0, _OC)], osems[b]
            ).wait()

    return k


def kernel(inputs, table):
    n_rows, seq = inputs.shape
    idx = inputs.reshape(inputs.size)
    table_p = jnp.pad(table, ((0, 0), (0, 2 * _DIM - table.shape[1])))
    return _make_gather(n_rows, seq)(table_p, idx)


# speedup vs baseline: 1.4753x; 1.1548x over previous
import functools

import jax
import jax.numpy as jnp
from jax import lax
from jax.experimental import pallas as pl
from jax.experimental.pallas import tpu as pltpu
from jax.experimental.pallas import tpu_sc as plsc

_DIM = 64
_SCALE = 10.0

_info = plsc.get_sparse_core_info()
_NC, _NS = _info.num_cores, _info.num_subcores
_NW = _NC * _NS

_OC = 4
_G1 = 104


@functools.lru_cache(maxsize=None)
def _make_gather(n_rows, seq):
    flat_per_chunk = _OC * seq
    assert n_rows % _NW == 0
    rows_per_w = n_rows // _NW
    assert rows_per_w % _OC == 0
    n_it = rows_per_w // _OC
    assert n_it % 2 == 0
    b_per_w = rows_per_w * seq
    g2 = flat_per_chunk - _G1
    mesh = plsc.VectorSubcoreMesh(core_axis_name="c", subcore_axis_name="s")

    @functools.partial(
        pl.kernel,
        mesh=mesh,
        out_type=jax.ShapeDtypeStruct((n_rows, seq, _DIM), jnp.float32),
        compiler_params=pltpu.CompilerParams(use_tc_tiling_on_sc=True),
        scratch_types=[
            pltpu.VMEM((b_per_w,), jnp.int32),
            pltpu.VMEM((2, flat_per_chunk, 2 * _DIM), jnp.float32),
            pltpu.VMEM((2, _OC, seq, _DIM), jnp.float32),
            pltpu.SemaphoreType.DMA,
            pltpu.SemaphoreType.DMA,
            pltpu.SemaphoreType.DMA,
            pltpu.SemaphoreType.DMA,
        ],
    )
    def k(table_hbm, idx_hbm, out_hbm, idx_v, rows, boxes, gsem0, gsem1,
          osem0, osem1):
        gsems = (gsem0, gsem1)
        osems = (osem0, osem1)
        wid = lax.axis_index("s") * _NC + lax.axis_index("c")
        base = wid * b_per_w
        row0 = wid * rows_per_w
        pltpu.sync_copy(idx_hbm.at[pl.ds(base, b_per_w)], idx_v)

        def fire_gathers(j, b):
            off = j * flat_per_chunk
            pltpu.async_copy(
                table_hbm.at[idx_v.at[pl.ds(off, _G1)]],
                rows.at[b, pl.ds(0, _G1)],
                gsems[b],
            )
            pltpu.async_copy(
                table_hbm.at[idx_v.at[pl.ds(off + _G1, g2)]],
                rows.at[b, pl.ds(_G1, g2)],
                gsems[b],
            )

        def wait_gathers(b):
            pltpu.make_async_copy(
                table_hbm.at[idx_v.at[pl.ds(0, _G1)]],
                rows.at[b, pl.ds(0, _G1)],
                gsems[b],
            ).wait()
            pltpu.make_async_copy(
                table_hbm.at[idx_v.at[pl.ds(0, g2)]],
                rows.at[b, pl.ds(_G1, g2)],
                gsems[b],
            ).wait()

        fire_gathers(0, 0)

        def grp(g, carry):
            for b in range(2):
                j = g * 2 + b
                nb = 1 - b
                wait_gathers(b)

                @pl.when(j + 1 < n_it)
                def _():
                    fire_gathers(j + 1, nb)

                @pl.when(j >= 2)
                def _():
                    pltpu.make_async_copy(
                        boxes.at[b], out_hbm.at[pl.ds(0, _OC)], osems[b]
                    ).wait()

                for o in range(_OC):
                    @plsc.parallel_loop(0, seq * _DIM // 16, step=1, unroll=8)
                    def _(p):
                        r = p // (_DIM // 16)
                        c = (p % (_DIM // 16)) * 16
                        boxes[b, o, r, pl.ds(c, 16)] = (
                            rows[b, o * seq + r, pl.ds(c, 16)] * _SCALE
                        )

                pltpu.async_copy(
                    boxes.at[b],
                    out_hbm.at[pl.ds(row0 + j * _OC, _OC)],
                    osems[b],
                )

            return carry

        lax.fori_loop(0, n_it // 2, grp, 0)
        for b in range(2):
            pltpu.make_async_copy(
                boxes.at[b], out_hbm.at[pl.ds(0, _OC)], osems[b]
            ).wait()

    return k


def kernel(inputs, table):
    n_rows, seq = inputs.shape
    idx = inputs.reshape(inputs.size)
    table_p = jnp.pad(table, ((0, 0), (0, 2 * _DIM - table.shape[1])))
    out = _make_gather(n_rows, seq)(table_p, idx)
    return jax.lax.optimization_barrier(out)
